# Initial kernel scaffold; baseline (speedup 1.0000x reference)
import jax
import jax.numpy as jnp
from jax.experimental import pallas as pl


def kernel(x, edge_index, batch, W1, b1, g1, be1, W2, b2, g2, be2, gc, bcn, Wc1, bc1, Wc2, bc2):
    raise NotImplementedError("write your pallas kernel here")


# speedup vs baseline: 3.2947x; 3.2947x over previous
import functools

import jax
import jax.numpy as jnp
from jax import lax
from jax.experimental import pallas as pl
from jax.experimental.pallas import tpu as pltpu
from jax.experimental.pallas import tpu_sc as plsc

_N = 10000
_E = 320000
_D = 128
_BLOCKS = 5
_G = 64
_C = 10

_NW = 32
_TILES = 16
_CHUNK = 128
_CPW = 79
_EPAD = _NW * _CPW * _CHUNK
_NPAD = 10240
_RPT = _NPAD // _TILES
_ZROWS = 64

_mesh = plsc.VectorSubcoreMesh(core_axis_name="c", subcore_axis_name="s")


@functools.partial(
    pl.kernel,
    mesh=_mesh,
    out_type=jax.ShapeDtypeStruct((2 * _NPAD, _D), jnp.float32),
    scratch_types=[
        pltpu.VMEM((_CHUNK,), jnp.int32),
        pltpu.VMEM((_CHUNK,), jnp.int32),
        pltpu.VMEM((_CHUNK, _D), jnp.float32),
        pltpu.VMEM((_ZROWS, _D), jnp.float32),
        pltpu.VMEM_SHARED((_NPAD, _D), jnp.float32),
        pltpu.SemaphoreType.DMA,
    ],
)
def _sc_agg(h_hbm, src_hbm, dst_hbm, out_hbm, sidx, didx, rows, zbuf, acc, sem):
    cid = lax.axis_index("c")
    sid = lax.axis_index("s")
    wid = sid * 2 + cid

    def _zstore(i, carry):
        r = i // 8
        col = (i % 8) * 16
        zbuf[r, pl.ds(col, 16)] = jnp.zeros((16,), jnp.float32)
        return carry

    lax.fori_loop(0, _ZROWS * 8, _zstore, 0)

    def _zcopy(k, carry):
        pltpu.sync_copy(zbuf, acc.at[pl.ds(sid * _RPT + k * _ZROWS, _ZROWS)])
        return carry

    lax.fori_loop(0, _RPT // _ZROWS, _zcopy, 0)
    plsc.subcore_barrier()

    def _edge_chunk(c, carry):
        base = pl.multiple_of(c * _CHUNK, 8)
        pltpu.sync_copy(src_hbm.at[pl.ds(base, _CHUNK)], sidx)
        pltpu.sync_copy(dst_hbm.at[pl.ds(base, _CHUNK)], didx)
        pltpu.async_copy(h_hbm.at[sidx], rows, sem).wait()
        pltpu.sync_copy(rows, acc.at[didx], add=True)
        return carry

    lax.fori_loop(wid * _CPW, (wid + 1) * _CPW, _edge_chunk, 0)
    plsc.subcore_barrier()

    pltpu.sync_copy(
        acc.at[pl.ds(sid * _RPT, _RPT)],
        out_hbm.at[pl.ds(cid * _NPAD + sid * _RPT, _RPT)],
    )


def _dense_body(h_ref, agg_ref, w1_ref, b1_ref, g1_ref, be1_ref,
                w2_ref, b2_ref, g2_ref, be2_ref, batch_ref,
                hout_ref, feat_ref):
    hin = h_ref[...] + agg_ref[0:_N, :] + agg_ref[_NPAD:_NPAD + _N, :]
    y = jnp.dot(hin, w1_ref[...], preferred_element_type=jnp.float32,
                precision=lax.Precision.HIGHEST) + b1_ref[...]
    y = jnp.maximum(y, 0.0)
    m = jnp.mean(y, axis=0, keepdims=True)
    v = jnp.mean((y - m) ** 2, axis=0, keepdims=True)
    y = (y - m) * lax.rsqrt(v + 1e-5) * g1_ref[...] + be1_ref[...]
    z = jnp.dot(y, w2_ref[...], preferred_element_type=jnp.float32,
                precision=lax.Precision.HIGHEST) + b2_ref[...]
    z = jnp.maximum(z, 0.0)
    m2 = jnp.mean(z, axis=0, keepdims=True)
    v2 = jnp.mean((z - m2) ** 2, axis=0, keepdims=True)
    z = (z - m2) * lax.rsqrt(v2 + 1e-5) * g2_ref[...] + be2_ref[...]
    hout_ref[...] = z
    onehot = (lax.broadcasted_iota(jnp.int32, (_G, _N), 0)
              == batch_ref[...]).astype(jnp.float32)
    feat_ref[...] = jnp.dot(onehot, z, preferred_element_type=jnp.float32,
                            precision=lax.Precision.HIGHEST)


_dense = pl.pallas_call(
    _dense_body,
    out_shape=(
        jax.ShapeDtypeStruct((_N, _D), jnp.float32),
        jax.ShapeDtypeStruct((_G, _D), jnp.float32),
    ),
)


def _cls_body(f_ref, gc_ref, bcn_ref, wc1_ref, bc1_ref, wc2_ref, bc2_ref,
              out_ref):
    f = f_ref[...]
    m = jnp.mean(f, axis=0, keepdims=True)
    v = jnp.mean((f - m) ** 2, axis=0, keepdims=True)
    f = (f - m) * lax.rsqrt(v + 1e-5) * gc_ref[...] + bcn_ref[...]
    z = jnp.dot(f, wc1_ref[...], preferred_element_type=jnp.float32,
                precision=lax.Precision.HIGHEST) + bc1_ref[...]
    z = jnp.maximum(z, 0.0)
    z = jnp.dot(z, wc2_ref[...], preferred_element_type=jnp.float32,
                precision=lax.Precision.HIGHEST) + bc2_ref[...]
    zmax = jnp.max(z, axis=-1, keepdims=True)
    lse = zmax + jnp.log(jnp.sum(jnp.exp(z - zmax), axis=-1, keepdims=True))
    out_ref[...] = z - lse


_classifier = pl.pallas_call(
    _cls_body,
    out_shape=jax.ShapeDtypeStruct((_G, _C), jnp.float32),
)


def kernel(x, edge_index, batch, W1, b1, g1, be1, W2, b2, g2, be2,
           gc, bcn, Wc1, bc1, Wc2, bc2):
    src = edge_index[0]
    dst = edge_index[1]
    pad = _EPAD - _E
    src_p = jnp.concatenate([src, jnp.zeros((pad,), jnp.int32)])
    dst_p = jnp.concatenate([dst, jnp.full((pad,), _N + 16, jnp.int32)])
    batch2d = batch.reshape(1, _N)

    h = x
    feats = []
    for i in range(_BLOCKS):
        agg = _sc_agg(h, src_p, dst_p)
        h, f = _dense(h, agg, W1[i], b1[i].reshape(1, _D), g1[i].reshape(1, _D),
                      be1[i].reshape(1, _D), W2[i], b2[i].reshape(1, _D),
                      g2[i].reshape(1, _D), be2[i].reshape(1, _D), batch2d)
        feats.append(f)
    fcat = jnp.concatenate(feats, axis=1)
    return _classifier(fcat, gc.reshape(1, _BLOCKS * _D),
                       bcn.reshape(1, _BLOCKS * _D), Wc1,
                       bc1.reshape(1, _D), Wc2, bc2.reshape(1, _C))
